# baseline (speedup 1.0000x reference)
import functools

import jax
import jax.numpy as jnp
from jax import lax
from jax.experimental import pallas as pl
from jax.experimental.pallas import tpu as pltpu
from jax.experimental.pallas import tpu_sc as plsc

_VOCAB = 1000
_BATCH = 1024
_SEQ = 50
_NTOK = _BATCH * _SEQ

_NC = 2
_NS = 16
_NW = _NC * _NS
_TOK_PER_W = _NTOK // _NW
_CHUNK = 40
_NCHUNK = _TOK_PER_W // _CHUNK


def _table_body(embed_ref, w_ref, b_ref, out_ref):
    out_ref[...] = lax.dot_general(
        embed_ref[...], w_ref[...],
        (((1,), (1,)), ((), ())),
        preferred_element_type=jnp.float32,
    ) + b_ref[...]


def _make_table(embed, W, b):
    return pl.pallas_call(
        _table_body,
        out_shape=jax.ShapeDtypeStruct((_VOCAB, _VOCAB), jnp.float32),
    )(embed, W, b.reshape(1, _VOCAB))


_sc_mesh = plsc.VectorSubcoreMesh(core_axis_name="c", subcore_axis_name="s")


@functools.partial(
    pl.kernel,
    out_type=jax.ShapeDtypeStruct((_NTOK, _VOCAB), jnp.float32),
    mesh=_sc_mesh,
    scratch_types=[
        pltpu.VMEM((_TOK_PER_W,), jnp.int32),
        pltpu.VMEM((_CHUNK, _VOCAB), jnp.float32),
        pltpu.SemaphoreType.DMA,
    ],
    compiler_params=pltpu.CompilerParams(use_tc_tiling_on_sc=False),
)
def _sc_gather(table_hbm, ids_hbm, out_hbm, idx_v, buf, sem):
    wid = lax.axis_index("s") * _NC + lax.axis_index("c")
    base = wid * _TOK_PER_W
    pltpu.sync_copy(ids_hbm.at[pl.ds(base, _TOK_PER_W)], idx_v)

    def body(g, carry):
        off = g * _CHUNK
        pltpu.async_copy(
            table_hbm.at[idx_v.at[pl.ds(off, _CHUNK)]], buf, sem
        ).wait()
        pltpu.sync_copy(buf, out_hbm.at[pl.ds(base + off, _CHUNK)])
        return carry

    lax.fori_loop(0, _NCHUNK, body, 0)


def kernel(input_ids, embed, W, b):
    table = _make_table(embed, W, b)
    flat_ids = input_ids.reshape(_NTOK).astype(jnp.int32)
    logits = _sc_gather(table, flat_ids)
    return logits.reshape(_BATCH, _SEQ, _VOCAB)


# speedup vs baseline: 1.0269x; 1.0269x over previous
import functools

import jax
import jax.numpy as jnp
from jax import lax
from jax.experimental import pallas as pl
from jax.experimental.pallas import tpu as pltpu
from jax.experimental.pallas import tpu_sc as plsc

_VOCAB = 1000
_EMB = 8
_BATCH = 1024
_SEQ = 50
_SEQ_PAD = 64

_NC = 2
_NS = 16
_NW = _NC * _NS
_BATCH_PER_W = _BATCH // _NW
_IDS_PER_W = _BATCH_PER_W * _SEQ_PAD


def _table_body(embed_ref, w_ref, b_ref, out_ref):
    out_ref[...] = lax.dot_general(
        embed_ref[...], w_ref[...],
        (((1,), (1,)), ((), ())),
        preferred_element_type=jnp.float32,
    ) + b_ref[...]


def _make_table(embed, W, b):
    return pl.pallas_call(
        _table_body,
        out_shape=jax.ShapeDtypeStruct((_VOCAB, _VOCAB), jnp.float32),
    )(embed, W, b.reshape(1, _VOCAB))


_sc_mesh = plsc.VectorSubcoreMesh(core_axis_name="c", subcore_axis_name="s")


@functools.partial(
    pl.kernel,
    out_type=jax.ShapeDtypeStruct((_BATCH, _SEQ, _VOCAB), jnp.float32),
    mesh=_sc_mesh,
    scratch_types=[
        pltpu.VMEM((_IDS_PER_W,), jnp.int32),
        pltpu.VMEM((_SEQ, _VOCAB), jnp.float32),
        pltpu.VMEM((_SEQ, _VOCAB), jnp.float32),
        pltpu.SemaphoreType.DMA,
        pltpu.SemaphoreType.DMA,
    ],
    compiler_params=pltpu.CompilerParams(use_tc_tiling_on_sc=False),
)
def _sc_gather(table_hbm, ids_hbm, out_hbm, idx_v, buf0, buf1, sem0, sem1):
    wid = lax.axis_index("s") * _NC + lax.axis_index("c")
    batch0 = wid * _BATCH_PER_W
    pltpu.sync_copy(ids_hbm.at[pl.ds(wid * _IDS_PER_W, _IDS_PER_W)], idx_v)

    def start(g, buf, sem):
        off = pl.multiple_of(g * _SEQ_PAD, 8)
        pltpu.async_copy(table_hbm.at[idx_v.at[pl.ds(off, _SEQ)]], buf, sem)

    def drain(g, buf, sem):
        pltpu.make_async_copy(
            table_hbm.at[idx_v.at[pl.ds(0, _SEQ)]], buf, sem
        ).wait()
        pltpu.sync_copy(buf, out_hbm.at[batch0 + g])

    start(0, buf0, sem0)

    def body(i, carry):
        g = 2 * i
        start(g + 1, buf1, sem1)
        drain(g, buf0, sem0)

        @pl.when(i < _BATCH_PER_W // 2 - 1)
        def _():
            start(g + 2, buf0, sem0)

        drain(g + 1, buf1, sem1)
        return carry

    lax.fori_loop(0, _BATCH_PER_W // 2, body, 0)


def kernel(input_ids, embed, W, b):
    table = _make_table(embed, W, b)
    ids_pad = jnp.pad(
        input_ids.astype(jnp.int32), ((0, 0), (0, _SEQ_PAD - _SEQ))
    ).reshape(_BATCH * _SEQ_PAD)
    return _sc_gather(table, ids_pad)
